# baseline (speedup 1.0000x reference)
import functools

import jax
import jax.numpy as jnp
from jax import lax
from jax.experimental import pallas as pl
from jax.experimental.pallas import tpu as pltpu
from jax.experimental.pallas import tpu_sc as plsc

NC = 2
NS = 16
NW = NC * NS
L = 16
CH = 128
BN = 1000


def _mesh():
    return plsc.VectorSubcoreMesh(core_axis_name="c", subcore_axis_name="s")


@functools.lru_cache(maxsize=None)
def _build_deg(N, E):
    R = E // CH
    SCH = 8
    nsc, rem = divmod(R, SCH)
    nt = (nsc + NW - 1) // NW
    ZB = 2000

    def body(src_h, dst_h, ew_h, outa_h, outb_h,
             srcb, dstb, ewb, ewpb, zb, degsp, is0, is1, ss0, ss1):
        isems = (is0, is1)
        ssems = (ss0, ss1)
        c = lax.axis_index("c")
        s = lax.axis_index("s")
        w = c * NS + s
        zero = jnp.zeros((L,), jnp.float32)

        @pl.when(s == 0)
        def _zero():
            def zb_body(i, _):
                zb[pl.ds(i * L, L)] = zero
                return 0
            lax.fori_loop(0, ZB // L, zb_body, 0)
            for t in range(N // ZB):
                pltpu.sync_copy(zb, degsp.at[pl.ds(t * ZB, ZB)])

        def issue_loads(row0_, isl):
            pltpu.async_copy(src_h.at[pl.ds(row0_, SCH)], srcb.at[isl],
                             isems[isl])
            pltpu.async_copy(dst_h.at[pl.ds(row0_, SCH)], dstb.at[isl],
                             isems[isl])
            pltpu.async_copy(ew_h.at[pl.ds(row0_, SCH)], ewb.at[isl],
                             isems[isl])

        def wait_loads(isl):
            pltpu.make_async_copy(src_h.at[pl.ds(0, SCH)], srcb.at[isl],
                                  isems[isl]).wait()
            pltpu.make_async_copy(dst_h.at[pl.ds(0, SCH)], dstb.at[isl],
                                  isems[isl]).wait()
            pltpu.make_async_copy(ew_h.at[pl.ds(0, SCH)], ewb.at[isl],
                                  isems[isl]).wait()

        def drain_super(P):
            for r in range(SCH):
                pltpu.make_async_copy(ew_h.at[0], ewpb.at[P, r],
                                      ssems[P]).wait()

        issue_loads(pl.multiple_of(w * SCH, 8), 0)
        plsc.subcore_barrier()

        def super_work(t, sc, P):
            @pl.when(t >= 2)
            def _():
                drain_super(P)
            wait_loads(P)
            for r in range(SCH):
                for g in range(CH // L):
                    sl = pl.ds(g * L, L)
                    ewpb[P, r, sl] = jnp.where(
                        srcb[P, r, sl] == dstb[P, r, sl], zero,
                        ewb[P, r, sl])
            for r in range(SCH):
                pltpu.async_copy(ewpb.at[P, r], degsp.at[srcb.at[P, r]],
                                 ssems[P], add=True)

            @pl.when((sc + NW) < nsc)
            def _():
                issue_loads(pl.multiple_of((sc + NW) * SCH, 8), 1 - P)

        def super_body(t, _):
            sc = w + NW * t

            @pl.when(sc < nsc)
            def _():
                slt = t % 2

                @pl.when(slt == 0)
                def _():
                    super_work(t, sc, 0)

                @pl.when(slt == 1)
                def _():
                    super_work(t, sc, 1)
            return 0
        lax.fori_loop(0, nt, super_body, 0)
        drain_super(0)
        drain_super(1)

        if rem:
            @pl.when(w == NW - 1)
            def _rem():
                b = pl.multiple_of(nsc * SCH, 8)
                pltpu.async_copy(src_h.at[pl.ds(b, rem)],
                                 srcb.at[0, pl.ds(0, rem)], isems[0])
                pltpu.async_copy(dst_h.at[pl.ds(b, rem)],
                                 dstb.at[0, pl.ds(0, rem)], isems[0])
                pltpu.async_copy(ew_h.at[pl.ds(b, rem)],
                                 ewb.at[0, pl.ds(0, rem)], isems[0])
                pltpu.make_async_copy(src_h.at[pl.ds(0, rem)],
                                      srcb.at[0, pl.ds(0, rem)],
                                      isems[0]).wait()
                pltpu.make_async_copy(dst_h.at[pl.ds(0, rem)],
                                      dstb.at[0, pl.ds(0, rem)],
                                      isems[0]).wait()
                pltpu.make_async_copy(ew_h.at[pl.ds(0, rem)],
                                      ewb.at[0, pl.ds(0, rem)],
                                      isems[0]).wait()
                for r in range(rem):
                    for g in range(CH // L):
                        sl = pl.ds(g * L, L)
                        ewpb[0, r, sl] = jnp.where(
                            srcb[0, r, sl] == dstb[0, r, sl], zero,
                            ewb[0, r, sl])
                    pltpu.sync_copy(ewpb.at[0, r], degsp.at[srcb.at[0, r]],
                                    add=True)

        plsc.subcore_barrier()

        @pl.when(jnp.logical_and(s == 0, c == 0))
        def _outa():
            pltpu.sync_copy(degsp, outa_h)

        @pl.when(jnp.logical_and(s == 0, c == 1))
        def _outb():
            pltpu.sync_copy(degsp, outb_h)

    return pl.kernel(
        body,
        out_type=(jax.ShapeDtypeStruct((N,), jnp.float32),
                  jax.ShapeDtypeStruct((N,), jnp.float32)),
        mesh=_mesh(),
        scratch_types=[
            pltpu.VMEM((2, SCH, CH), jnp.int32),
            pltpu.VMEM((2, SCH, CH), jnp.int32),
            pltpu.VMEM((2, SCH, CH), jnp.float32),
            pltpu.VMEM((2, SCH, CH), jnp.float32),
            pltpu.VMEM((ZB,), jnp.float32),
            pltpu.VMEM_SHARED((N,), jnp.float32),
            pltpu.SemaphoreType.DMA,
            pltpu.SemaphoreType.DMA,
            pltpu.SemaphoreType.DMA,
            pltpu.SemaphoreType.DMA,
        ],
    )


@functools.lru_cache(maxsize=None)
def _build_prop(N, F, E):
    R = E // CH
    SCH = 8
    nsc, rem = divmod(R, SCH)
    nt = (nsc + NW - 1) // NW
    n_rch, r_tail = divmod(N, CH)
    nz = (n_rch + NS - 1) // NS

    def body(h_h, src_h, dst_h, ew_h, out_h,
             srcb, dstb, ewb, rows0, rows1, acc,
             is0, is1, gs0, gs1, ss0, ss1):
        rowss = (rows0, rows1)
        isems = (is0, is1)
        gsems = (gs0, gs1)
        ssems = (ss0, ss1)
        c = lax.axis_index("c")
        s = lax.axis_index("s")
        w = c * NS + s
        zero = jnp.zeros((L,), jnp.float32)

        def zr(i, _):
            for f in range(F // L):
                rows0[i, pl.ds(f * L, L)] = zero
            return 0
        lax.fori_loop(0, CH, zr, 0)
        for t in range(nz):
            j = s + t * NS

            @pl.when(j < n_rch)
            def _():
                pltpu.sync_copy(rows0,
                                acc.at[pl.ds(pl.multiple_of(j * CH, 8), CH)])
        if r_tail:
            @pl.when(s == 0)
            def _ztail():
                pltpu.sync_copy(rows0.at[pl.ds(0, r_tail)],
                                acc.at[pl.ds(n_rch * CH, r_tail)])

        def issue_loads(row0_, isl):
            pltpu.async_copy(src_h.at[pl.ds(row0_, SCH)], srcb.at[isl],
                             isems[isl])
            pltpu.async_copy(dst_h.at[pl.ds(row0_, SCH)], dstb.at[isl],
                             isems[isl])
            pltpu.async_copy(ew_h.at[pl.ds(row0_, SCH)], ewb.at[isl],
                             isems[isl])

        def wait_loads(isl):
            pltpu.make_async_copy(src_h.at[pl.ds(0, SCH)], srcb.at[isl],
                                  isems[isl]).wait()
            pltpu.make_async_copy(dst_h.at[pl.ds(0, SCH)], dstb.at[isl],
                                  isems[isl]).wait()
            pltpu.make_async_copy(ew_h.at[pl.ds(0, SCH)], ewb.at[isl],
                                  isems[isl]).wait()

        def sem_wait_rows(semlist, A):
            pltpu.make_async_copy(h_h.at[pl.ds(0, CH)], rowss[A],
                                  semlist[A]).wait()

        def scale(A, slt, r):
            rb = rowss[A]

            def grp(g, _):
                sl = pl.ds(g * L, L)
                nv = jnp.where(srcb[slt, r, sl] == dstb[slt, r, sl],
                               zero, -ewb[slt, r, sl])
                for i16 in range(L):
                    i = g * L + i16
                    sv = jnp.full((L,), nv[i16], jnp.float32)
                    for f in range(F // L):
                        fs = pl.ds(f * L, L)
                        rb[i, fs] = rb[i, fs] * sv
                return 0
            lax.fori_loop(0, CH // L, grp, 0)

        issue_loads(pl.multiple_of(w * SCH, 8), 0)
        wait_loads(0)
        pltpu.async_copy(h_h.at[srcb.at[0, 0]], rowss[0], gsems[0])
        plsc.subcore_barrier()

        def super_body(t, _):
            sc = w + NW * t

            @pl.when(sc < nsc)
            def _():
                slt = t % 2
                nxt = pl.multiple_of((sc + NW) * SCH, 8)
                have_next = (sc + NW) < nsc
                for r in range(SCH):
                    A = r % 2
                    B = 1 - A
                    sem_wait_rows(gsems, A)
                    if r == 0:
                        @pl.when(t > 0)
                        def _():
                            sem_wait_rows(ssems, B)

                        @pl.when(jnp.logical_and(have_next, slt == 0))
                        def _():
                            issue_loads(nxt, 1)

                        @pl.when(jnp.logical_and(have_next, slt == 1))
                        def _():
                            issue_loads(nxt, 0)
                    else:
                        sem_wait_rows(ssems, B)
                    if r < SCH - 1:
                        pltpu.async_copy(h_h.at[srcb.at[slt, r + 1]],
                                         rowss[B], gsems[B])
                    else:
                        @pl.when(jnp.logical_and(have_next, slt == 0))
                        def _():
                            wait_loads(1)
                            pltpu.async_copy(h_h.at[srcb.at[1, 0]],
                                             rowss[B], gsems[B])

                        @pl.when(jnp.logical_and(have_next, slt == 1))
                        def _():
                            wait_loads(0)
                            pltpu.async_copy(h_h.at[srcb.at[0, 0]],
                                             rowss[B], gsems[B])
                    scale(A, slt, r)
                    pltpu.async_copy(rowss[A], acc.at[dstb.at[slt, r]],
                                     ssems[A], add=True)
            return 0
        lax.fori_loop(0, nt, super_body, 0)
        sem_wait_rows(ssems, 1)

        if rem:
            @pl.when(w == NW - 1)
            def _rem():
                b = pl.multiple_of(nsc * SCH, 8)
                pltpu.async_copy(src_h.at[pl.ds(b, rem)],
                                 srcb.at[0, pl.ds(0, rem)], isems[0])
                pltpu.async_copy(dst_h.at[pl.ds(b, rem)],
                                 dstb.at[0, pl.ds(0, rem)], isems[0])
                pltpu.async_copy(ew_h.at[pl.ds(b, rem)],
                                 ewb.at[0, pl.ds(0, rem)], isems[0])
                pltpu.make_async_copy(src_h.at[pl.ds(0, rem)],
                                      srcb.at[0, pl.ds(0, rem)],
                                      isems[0]).wait()
                pltpu.make_async_copy(dst_h.at[pl.ds(0, rem)],
                                      dstb.at[0, pl.ds(0, rem)],
                                      isems[0]).wait()
                pltpu.make_async_copy(ew_h.at[pl.ds(0, rem)],
                                      ewb.at[0, pl.ds(0, rem)],
                                      isems[0]).wait()

                def remrow(rr, _):
                    pltpu.async_copy(h_h.at[srcb.at[0, rr]], rowss[0],
                                     gsems[0]).wait()
                    scale(0, 0, rr)
                    pltpu.sync_copy(rowss[0], acc.at[dstb.at[0, rr]],
                                    add=True)
                    return 0
                lax.fori_loop(0, rem, remrow, 0)

        plsc.subcore_barrier()
        for t in range(nz):
            j = s + t * NS

            @pl.when(j < n_rch)
            def _():
                off = pl.multiple_of(j * CH, 8)
                pltpu.sync_copy(acc.at[pl.ds(off, CH)],
                                out_h.at[c, pl.ds(off, CH)])
        if r_tail:
            @pl.when(s == 0)
            def _etail():
                pltpu.sync_copy(acc.at[pl.ds(n_rch * CH, r_tail)],
                                out_h.at[c, pl.ds(n_rch * CH, r_tail)])

    return pl.kernel(
        body,
        out_type=jax.ShapeDtypeStruct((NC, N, F), jnp.float32),
        mesh=_mesh(),
        scratch_types=(
            [pltpu.VMEM((2, SCH, CH), jnp.int32)] * 2
            + [pltpu.VMEM((2, SCH, CH), jnp.float32)]
            + [pltpu.VMEM((CH, F), jnp.float32)] * 2
            + [pltpu.VMEM_SHARED((N, F), jnp.float32)]
            + [pltpu.SemaphoreType.DMA] * 6
        ),
    )


def _tc_prep(degpT, x, W0):
    N, F = x.shape
    FO = W0.shape[1]

    def body(dp_ref, x_ref, w0_ref, dis_ref, h2_ref, out_ref):
        deg = dp_ref[:, 0:1] + dp_ref[:, 1:2]
        pos = deg > 0.0
        dis = jnp.where(pos, lax.rsqrt(jnp.where(pos, deg, 1.0)), 0.0)
        dis_ref[...] = dis
        h2_ref[...] = x_ref[...] * dis
        out_ref[...] = jnp.dot(x_ref[...], w0_ref[...],
                               preferred_element_type=jnp.float32)

    return pl.pallas_call(
        body,
        grid=(N // BN,),
        in_specs=[pl.BlockSpec((BN, 2), lambda i: (i, 0)),
                  pl.BlockSpec((BN, F), lambda i: (i, 0)),
                  pl.BlockSpec((F, FO), lambda i: (0, 0))],
        out_specs=[pl.BlockSpec((BN, 1), lambda i: (i, 0)),
                   pl.BlockSpec((BN, F), lambda i: (i, 0)),
                   pl.BlockSpec((BN, FO), lambda i: (i, 0))],
        out_shape=[jax.ShapeDtypeStruct((N, 1), jnp.float32),
                   jax.ShapeDtypeStruct((N, F), jnp.float32),
                   jax.ShapeDtypeStruct((N, FO), jnp.float32)],
    )(degpT, x, W0)


def _tc_step_ew(dis, pA, pB, txpp, first):
    N, F = pA.shape

    def body(dis_ref, pa_ref, pb_ref, txpp_ref, tx_ref, h2_ref):
        dis = dis_ref[...]
        t = dis * (pa_ref[...] + pb_ref[...])
        if first:
            tx = t
        else:
            tx = 2.0 * t - txpp_ref[...]
        tx_ref[...] = tx
        h2_ref[...] = dis * tx

    return pl.pallas_call(
        body,
        grid=(N // BN,),
        in_specs=[
            pl.BlockSpec((BN, 1), lambda i: (i, 0)),
            pl.BlockSpec((BN, F), lambda i: (i, 0)),
            pl.BlockSpec((BN, F), lambda i: (i, 0)),
            pl.BlockSpec((BN, F), lambda i: (i, 0)),
        ],
        out_specs=[pl.BlockSpec((BN, F), lambda i: (i, 0)),
                   pl.BlockSpec((BN, F), lambda i: (i, 0))],
        out_shape=[jax.ShapeDtypeStruct((N, F), jnp.float32),
                   jax.ShapeDtypeStruct((N, F), jnp.float32)],
    )(dis, pA, pB, txpp)


def _tc_step_mm(tx, W, out_in, bias, last):
    N, F = tx.shape
    FO = W.shape[1]

    def body(tx_ref, w_ref, oin_ref, b_ref, out_ref):
        o = oin_ref[...] + jnp.dot(tx_ref[...], w_ref[...],
                                   preferred_element_type=jnp.float32)
        if last:
            o = jnp.maximum(o + b_ref[...], 0.0)
        out_ref[...] = o

    return pl.pallas_call(
        body,
        grid=(N // BN,),
        in_specs=[
            pl.BlockSpec((BN, F), lambda i: (i, 0)),
            pl.BlockSpec((F, FO), lambda i: (0, 0)),
            pl.BlockSpec((BN, FO), lambda i: (i, 0)),
            pl.BlockSpec((1, FO), lambda i: (0, 0)),
        ],
        out_specs=[pl.BlockSpec((BN, FO), lambda i: (i, 0))],
        out_shape=[jax.ShapeDtypeStruct((N, FO), jnp.float32)],
    )(tx, W, out_in, bias)[0]


def kernel(x, edge_index, edge_weight, Ws, bias):
    N, F = x.shape
    E = edge_weight.shape[0]
    K = Ws.shape[0]
    src = edge_index[0]
    dst = edge_index[1]
    bias2d = bias.reshape(1, -1)

    deg_fn = _build_deg(N, E)
    prop_fn = _build_prop(N, F, E)

    src2 = src.reshape(E // CH, CH)
    dst2 = dst.reshape(E // CH, CH)
    ew2 = edge_weight.reshape(E // CH, CH)
    degA, degB = deg_fn(src2, dst2, ew2)
    dis, h2, out = _tc_prep(jnp.stack([degA, degB], axis=1), x, Ws[0])

    tx_prev = x
    tx_pp = x
    for k in range(1, K):
        p = prop_fn(h2, src2, dst2, ew2)
        tx_new, h2 = _tc_step_ew(dis, p[0], p[1], tx_pp, first=(k == 1))
        out = _tc_step_mm(tx_new, Ws[k], out, bias2d, last=(k == K - 1))
        tx_pp, tx_prev = tx_prev, tx_new
    return out


# speedup vs baseline: 1.0087x; 1.0087x over previous
import functools

import jax
import jax.numpy as jnp
from jax import lax
from jax.experimental import pallas as pl
from jax.experimental.pallas import tpu as pltpu
from jax.experimental.pallas import tpu_sc as plsc

NC = 2
NS = 16
NW = NC * NS
L = 16
CH = 128
BN = 1000


def _mesh():
    return plsc.VectorSubcoreMesh(core_axis_name="c", subcore_axis_name="s")


@functools.lru_cache(maxsize=None)
def _build_deg(N, E):
    R = E // CH
    SCH = 8
    nsc, rem = divmod(R, SCH)
    nt = (nsc + NW - 1) // NW
    ZB = 2000

    def body(src_h, dst_h, ew_h, outa_h, outb_h,
             srcb, dstb, ewb, ewpb, zb, degsp, is0, is1, ss0, ss1):
        isems = (is0, is1)
        ssems = (ss0, ss1)
        c = lax.axis_index("c")
        s = lax.axis_index("s")
        w = c * NS + s
        zero = jnp.zeros((L,), jnp.float32)

        @pl.when(s == 0)
        def _zero():
            def zb_body(i, _):
                zb[pl.ds(i * L, L)] = zero
                return 0
            lax.fori_loop(0, ZB // L, zb_body, 0)
            for t in range(N // ZB):
                pltpu.sync_copy(zb, degsp.at[pl.ds(t * ZB, ZB)])

        def issue_loads(row0_, isl):
            pltpu.async_copy(src_h.at[pl.ds(row0_, SCH)], srcb.at[isl],
                             isems[isl])
            pltpu.async_copy(dst_h.at[pl.ds(row0_, SCH)], dstb.at[isl],
                             isems[isl])
            pltpu.async_copy(ew_h.at[pl.ds(row0_, SCH)], ewb.at[isl],
                             isems[isl])

        def wait_loads(isl):
            pltpu.make_async_copy(src_h.at[pl.ds(0, SCH)], srcb.at[isl],
                                  isems[isl]).wait()
            pltpu.make_async_copy(dst_h.at[pl.ds(0, SCH)], dstb.at[isl],
                                  isems[isl]).wait()
            pltpu.make_async_copy(ew_h.at[pl.ds(0, SCH)], ewb.at[isl],
                                  isems[isl]).wait()

        def drain_super(P):
            for r in range(SCH):
                pltpu.make_async_copy(ew_h.at[0], ewpb.at[P, r],
                                      ssems[P]).wait()

        issue_loads(pl.multiple_of(w * SCH, 8), 0)
        plsc.subcore_barrier()

        def super_work(t, sc, P):
            @pl.when(t >= 2)
            def _():
                drain_super(P)
            wait_loads(P)
            for r in range(SCH):
                for g in range(CH // L):
                    sl = pl.ds(g * L, L)
                    ewpb[P, r, sl] = jnp.where(
                        srcb[P, r, sl] == dstb[P, r, sl], zero,
                        ewb[P, r, sl])
            for r in range(SCH):
                pltpu.async_copy(ewpb.at[P, r], degsp.at[srcb.at[P, r]],
                                 ssems[P], add=True)

            @pl.when((sc + NW) < nsc)
            def _():
                issue_loads(pl.multiple_of((sc + NW) * SCH, 8), 1 - P)

        def super_body(t, _):
            sc = w + NW * t

            @pl.when(sc < nsc)
            def _():
                slt = t % 2

                @pl.when(slt == 0)
                def _():
                    super_work(t, sc, 0)

                @pl.when(slt == 1)
                def _():
                    super_work(t, sc, 1)
            return 0
        lax.fori_loop(0, nt, super_body, 0)
        drain_super(0)
        drain_super(1)

        if rem:
            @pl.when(w == NW - 1)
            def _rem():
                b = pl.multiple_of(nsc * SCH, 8)
                pltpu.async_copy(src_h.at[pl.ds(b, rem)],
                                 srcb.at[0, pl.ds(0, rem)], isems[0])
                pltpu.async_copy(dst_h.at[pl.ds(b, rem)],
                                 dstb.at[0, pl.ds(0, rem)], isems[0])
                pltpu.async_copy(ew_h.at[pl.ds(b, rem)],
                                 ewb.at[0, pl.ds(0, rem)], isems[0])
                pltpu.make_async_copy(src_h.at[pl.ds(0, rem)],
                                      srcb.at[0, pl.ds(0, rem)],
                                      isems[0]).wait()
                pltpu.make_async_copy(dst_h.at[pl.ds(0, rem)],
                                      dstb.at[0, pl.ds(0, rem)],
                                      isems[0]).wait()
                pltpu.make_async_copy(ew_h.at[pl.ds(0, rem)],
                                      ewb.at[0, pl.ds(0, rem)],
                                      isems[0]).wait()
                for r in range(rem):
                    for g in range(CH // L):
                        sl = pl.ds(g * L, L)
                        ewpb[0, r, sl] = jnp.where(
                            srcb[0, r, sl] == dstb[0, r, sl], zero,
                            ewb[0, r, sl])
                    pltpu.sync_copy(ewpb.at[0, r], degsp.at[srcb.at[0, r]],
                                    add=True)

        plsc.subcore_barrier()

        @pl.when(jnp.logical_and(s == 0, c == 0))
        def _outa():
            pltpu.sync_copy(degsp, outa_h)

        @pl.when(jnp.logical_and(s == 0, c == 1))
        def _outb():
            pltpu.sync_copy(degsp, outb_h)

    return pl.kernel(
        body,
        out_type=(jax.ShapeDtypeStruct((N,), jnp.float32),
                  jax.ShapeDtypeStruct((N,), jnp.float32)),
        mesh=_mesh(),
        scratch_types=[
            pltpu.VMEM((2, SCH, CH), jnp.int32),
            pltpu.VMEM((2, SCH, CH), jnp.int32),
            pltpu.VMEM((2, SCH, CH), jnp.float32),
            pltpu.VMEM((2, SCH, CH), jnp.float32),
            pltpu.VMEM((ZB,), jnp.float32),
            pltpu.VMEM_SHARED((N,), jnp.float32),
            pltpu.SemaphoreType.DMA,
            pltpu.SemaphoreType.DMA,
            pltpu.SemaphoreType.DMA,
            pltpu.SemaphoreType.DMA,
        ],
    )


@functools.lru_cache(maxsize=None)
def _build_prop(N, F, E):
    R = E // CH
    SCH = 8
    nsc, rem = divmod(R, SCH)
    nt = (nsc + NW - 1) // NW
    n_rch, r_tail = divmod(N, CH)
    nz = (n_rch + NS - 1) // NS

    def body(h_h, src_h, dst_h, ew_h, out_h,
             srcb, dstb, ewb, rows0, rows1, acc,
             is0, is1, gs0, gs1, ss0, ss1):
        rowss = (rows0, rows1)
        isems = (is0, is1)
        gsems = (gs0, gs1)
        ssems = (ss0, ss1)
        c = lax.axis_index("c")
        s = lax.axis_index("s")
        w = c * NS + s
        zero = jnp.zeros((L,), jnp.float32)

        def zr(i, _):
            for f in range(F // L):
                rows0[i, pl.ds(f * L, L)] = zero
            return 0
        lax.fori_loop(0, CH, zr, 0)
        for t in range(nz):
            j = s + t * NS

            @pl.when(j < n_rch)
            def _():
                pltpu.sync_copy(rows0,
                                acc.at[pl.ds(pl.multiple_of(j * CH, 8), CH)])
        if r_tail:
            @pl.when(s == 0)
            def _ztail():
                pltpu.sync_copy(rows0.at[pl.ds(0, r_tail)],
                                acc.at[pl.ds(n_rch * CH, r_tail)])

        def issue_loads(row0_, isl):
            pltpu.async_copy(src_h.at[pl.ds(row0_, SCH)], srcb.at[isl],
                             isems[isl])
            pltpu.async_copy(dst_h.at[pl.ds(row0_, SCH)], dstb.at[isl],
                             isems[isl])
            pltpu.async_copy(ew_h.at[pl.ds(row0_, SCH)], ewb.at[isl],
                             isems[isl])

        def wait_loads(isl):
            pltpu.make_async_copy(src_h.at[pl.ds(0, SCH)], srcb.at[isl],
                                  isems[isl]).wait()
            pltpu.make_async_copy(dst_h.at[pl.ds(0, SCH)], dstb.at[isl],
                                  isems[isl]).wait()
            pltpu.make_async_copy(ew_h.at[pl.ds(0, SCH)], ewb.at[isl],
                                  isems[isl]).wait()

        def sem_wait_rows(semlist, A):
            pltpu.make_async_copy(h_h.at[pl.ds(0, CH)], rowss[A],
                                  semlist[A]).wait()

        def scale(A, slt, r):
            rb = rowss[A]

            def grp(g, _):
                sl = pl.ds(g * L, L)
                nv = jnp.where(srcb[slt, r, sl] == dstb[slt, r, sl],
                               zero, -ewb[slt, r, sl])
                for i16 in range(L):
                    i = g * L + i16
                    sv = jnp.full((L,), nv[i16], jnp.float32)
                    for f in range(F // L):
                        fs = pl.ds(f * L, L)
                        rb[i, fs] = rb[i, fs] * sv
                return 0
            lax.fori_loop(0, CH // L, grp, 0)

        issue_loads(pl.multiple_of(w * SCH, 8), 0)
        wait_loads(0)
        pltpu.async_copy(h_h.at[srcb.at[0, 0]], rowss[0], gsems[0])
        plsc.subcore_barrier()

        def super_body(t, _):
            sc = w + NW * t

            @pl.when(sc < nsc)
            def _():
                slt = t % 2
                nxt = pl.multiple_of((sc + NW) * SCH, 8)
                have_next = (sc + NW) < nsc
                for r in range(SCH):
                    A = r % 2
                    B = 1 - A
                    sem_wait_rows(gsems, A)
                    if r == 0:
                        @pl.when(t > 0)
                        def _():
                            sem_wait_rows(ssems, B)

                        @pl.when(jnp.logical_and(have_next, slt == 0))
                        def _():
                            issue_loads(nxt, 1)

                        @pl.when(jnp.logical_and(have_next, slt == 1))
                        def _():
                            issue_loads(nxt, 0)
                    else:
                        sem_wait_rows(ssems, B)
                    if r < SCH - 1:
                        pltpu.async_copy(h_h.at[srcb.at[slt, r + 1]],
                                         rowss[B], gsems[B])
                    else:
                        @pl.when(jnp.logical_and(have_next, slt == 0))
                        def _():
                            wait_loads(1)
                            pltpu.async_copy(h_h.at[srcb.at[1, 0]],
                                             rowss[B], gsems[B])

                        @pl.when(jnp.logical_and(have_next, slt == 1))
                        def _():
                            wait_loads(0)
                            pltpu.async_copy(h_h.at[srcb.at[0, 0]],
                                             rowss[B], gsems[B])
                    scale(A, slt, r)
                    pltpu.async_copy(rowss[A], acc.at[dstb.at[slt, r]],
                                     ssems[A], add=True)
            return 0
        lax.fori_loop(0, nt, super_body, 0)
        sem_wait_rows(ssems, 1)

        if rem:
            @pl.when(w == NW - 1)
            def _rem():
                b = pl.multiple_of(nsc * SCH, 8)
                pltpu.async_copy(src_h.at[pl.ds(b, rem)],
                                 srcb.at[0, pl.ds(0, rem)], isems[0])
                pltpu.async_copy(dst_h.at[pl.ds(b, rem)],
                                 dstb.at[0, pl.ds(0, rem)], isems[0])
                pltpu.async_copy(ew_h.at[pl.ds(b, rem)],
                                 ewb.at[0, pl.ds(0, rem)], isems[0])
                pltpu.make_async_copy(src_h.at[pl.ds(0, rem)],
                                      srcb.at[0, pl.ds(0, rem)],
                                      isems[0]).wait()
                pltpu.make_async_copy(dst_h.at[pl.ds(0, rem)],
                                      dstb.at[0, pl.ds(0, rem)],
                                      isems[0]).wait()
                pltpu.make_async_copy(ew_h.at[pl.ds(0, rem)],
                                      ewb.at[0, pl.ds(0, rem)],
                                      isems[0]).wait()

                def remrow(rr, _):
                    pltpu.async_copy(h_h.at[srcb.at[0, rr]], rowss[0],
                                     gsems[0]).wait()
                    scale(0, 0, rr)
                    pltpu.sync_copy(rowss[0], acc.at[dstb.at[0, rr]],
                                    add=True)
                    return 0
                lax.fori_loop(0, rem, remrow, 0)

        plsc.subcore_barrier()
        for t in range(nz):
            j = s + t * NS

            @pl.when(j < n_rch)
            def _():
                off = pl.multiple_of(j * CH, 8)
                pltpu.sync_copy(acc.at[pl.ds(off, CH)],
                                out_h.at[c, pl.ds(off, CH)])
        if r_tail:
            @pl.when(s == 0)
            def _etail():
                pltpu.sync_copy(acc.at[pl.ds(n_rch * CH, r_tail)],
                                out_h.at[c, pl.ds(n_rch * CH, r_tail)])

    return pl.kernel(
        body,
        out_type=jax.ShapeDtypeStruct((NC, N, F), jnp.float32),
        mesh=_mesh(),
        scratch_types=(
            [pltpu.VMEM((2, SCH, CH), jnp.int32)] * 2
            + [pltpu.VMEM((2, SCH, CH), jnp.float32)]
            + [pltpu.VMEM((CH, F), jnp.float32)] * 2
            + [pltpu.VMEM_SHARED((N, F), jnp.float32)]
            + [pltpu.SemaphoreType.DMA] * 6
        ),
    )


def _tc_prep(degpT, x):
    N, F = x.shape

    def body(dp_ref, x_ref, dis_ref, h2_ref):
        deg = dp_ref[:, 0:1] + dp_ref[:, 1:2]
        pos = deg > 0.0
        dis = jnp.where(pos, lax.rsqrt(jnp.where(pos, deg, 1.0)), 0.0)
        dis_ref[...] = dis
        h2_ref[...] = x_ref[...] * dis

    return pl.pallas_call(
        body,
        grid=(N // BN,),
        in_specs=[pl.BlockSpec((BN, 2), lambda i: (i, 0)),
                  pl.BlockSpec((BN, F), lambda i: (i, 0))],
        out_specs=[pl.BlockSpec((BN, 1), lambda i: (i, 0)),
                   pl.BlockSpec((BN, F), lambda i: (i, 0))],
        out_shape=[jax.ShapeDtypeStruct((N, 1), jnp.float32),
                   jax.ShapeDtypeStruct((N, F), jnp.float32)],
    )(degpT, x)


def _tc_first(x, dis, pA, pB, W0, W1):
    N, F = x.shape
    FO = W0.shape[1]

    def body(x_ref, dis_ref, pa_ref, pb_ref, w0_ref, w1_ref,
             tx_ref, h2_ref, out_ref):
        dis = dis_ref[...]
        tx = dis * (pa_ref[...] + pb_ref[...])
        tx_ref[...] = tx
        h2_ref[...] = dis * tx
        out_ref[...] = (
            jnp.dot(x_ref[...], w0_ref[...], preferred_element_type=jnp.float32)
            + jnp.dot(tx, w1_ref[...], preferred_element_type=jnp.float32))

    return pl.pallas_call(
        body,
        grid=(N // BN,),
        in_specs=[
            pl.BlockSpec((BN, F), lambda i: (i, 0)),
            pl.BlockSpec((BN, 1), lambda i: (i, 0)),
            pl.BlockSpec((BN, F), lambda i: (i, 0)),
            pl.BlockSpec((BN, F), lambda i: (i, 0)),
            pl.BlockSpec((F, FO), lambda i: (0, 0)),
            pl.BlockSpec((F, FO), lambda i: (0, 0)),
        ],
        out_specs=[pl.BlockSpec((BN, F), lambda i: (i, 0)),
                   pl.BlockSpec((BN, F), lambda i: (i, 0)),
                   pl.BlockSpec((BN, FO), lambda i: (i, 0))],
        out_shape=[jax.ShapeDtypeStruct((N, F), jnp.float32),
                   jax.ShapeDtypeStruct((N, F), jnp.float32),
                   jax.ShapeDtypeStruct((N, FO), jnp.float32)],
    )(x, dis, pA, pB, W0, W1)


def _tc_step(dis, pA, pB, txpp, W, out_in, bias, last):
    N, F = pA.shape
    FO = W.shape[1]

    def body(dis_ref, pa_ref, pb_ref, txpp_ref, w_ref, oin_ref, b_ref,
             tx_ref, h2_ref, out_ref):
        dis = dis_ref[...]
        tx = 2.0 * dis * (pa_ref[...] + pb_ref[...]) - txpp_ref[...]
        tx_ref[...] = tx
        h2_ref[...] = dis * tx
        o = oin_ref[...] + jnp.dot(tx, w_ref[...],
                                   preferred_element_type=jnp.float32)
        if last:
            o = jnp.maximum(o + b_ref[...], 0.0)
        out_ref[...] = o

    return pl.pallas_call(
        body,
        grid=(N // BN,),
        in_specs=[
            pl.BlockSpec((BN, 1), lambda i: (i, 0)),
            pl.BlockSpec((BN, F), lambda i: (i, 0)),
            pl.BlockSpec((BN, F), lambda i: (i, 0)),
            pl.BlockSpec((BN, F), lambda i: (i, 0)),
            pl.BlockSpec((F, FO), lambda i: (0, 0)),
            pl.BlockSpec((BN, FO), lambda i: (i, 0)),
            pl.BlockSpec((1, FO), lambda i: (0, 0)),
        ],
        out_specs=[pl.BlockSpec((BN, F), lambda i: (i, 0)),
                   pl.BlockSpec((BN, F), lambda i: (i, 0)),
                   pl.BlockSpec((BN, FO), lambda i: (i, 0))],
        out_shape=[jax.ShapeDtypeStruct((N, F), jnp.float32),
                   jax.ShapeDtypeStruct((N, F), jnp.float32),
                   jax.ShapeDtypeStruct((N, FO), jnp.float32)],
    )(dis, pA, pB, txpp, W, out_in, bias)


def kernel(x, edge_index, edge_weight, Ws, bias):
    N, F = x.shape
    E = edge_weight.shape[0]
    K = Ws.shape[0]
    src = edge_index[0]
    dst = edge_index[1]
    bias2d = bias.reshape(1, -1)

    deg_fn = _build_deg(N, E)
    prop_fn = _build_prop(N, F, E)

    src2 = src.reshape(E // CH, CH)
    dst2 = dst.reshape(E // CH, CH)
    ew2 = edge_weight.reshape(E // CH, CH)
    degA, degB = deg_fn(src2, dst2, ew2)
    dis, h2 = _tc_prep(jnp.stack([degA, degB], axis=1), x)

    p = prop_fn(h2, src2, dst2, ew2)
    tx_prev, h2, out = _tc_first(x, dis, p[0], p[1], Ws[0], Ws[1])
    tx_pp = x
    for k in range(2, K):
        p = prop_fn(h2, src2, dst2, ew2)
        tx_new, h2, out = _tc_step(dis, p[0], p[1], tx_pp, Ws[k], out, bias2d,
                                   last=(k == K - 1))
        tx_pp, tx_prev = tx_prev, tx_new
    return out
